# baseline (speedup 1.0000x reference)
import functools

import jax
import jax.numpy as jnp
from jax import lax
from jax.experimental import pallas as pl
from jax.experimental.pallas import tpu as pltpu
from jax.experimental.pallas import tpu_sc as plsc

_D = 1024
_N = 8192
_S = 8
_B = 64
_K = 8

_NBLK = 512
_GRID = _N // _NBLK
_NCAND = _GRID * _K


def _topk8(work, cols):
    vs, ids = [], []
    for k in range(_K):
        m = jnp.max(work, axis=1, keepdims=True)
        pos = jnp.min(jnp.where(work == m, cols, jnp.int32(2**30)),
                      axis=1, keepdims=True)
        vs.append(m)
        ids.append(pos)
        work = jnp.where(cols == pos, -jnp.inf, work)
    return jnp.concatenate(vs, axis=1), jnp.concatenate(ids, axis=1)


def _tc_body(docs_ref, q_ref, wq_ref, bq_ref, wdoc_ref, bdoc_ref,
             vals_ref, idx_ref, qn_ref, cvals_ref, cidx_ref):
    i = pl.program_id(0)

    @pl.when(i == 0)
    def _():
        q = lax.dot_general(q_ref[...], wq_ref[...],
                            (((1,), (1,)), ((), ()))) + bq_ref[...]
        n2 = jnp.sum(q * q, axis=1, keepdims=True)
        qn_ref[...] = q / jnp.maximum(jnp.sqrt(n2), 1e-8)

    avg = jnp.sum(docs_ref[...], axis=1) * (1.0 / _S)

    demb = lax.dot_general(avg, wdoc_ref[...],
                           (((1,), (1,)), ((), ()))) + bdoc_ref[...]
    n2 = jnp.sum(demb * demb, axis=1, keepdims=True)
    demb = demb / jnp.maximum(jnp.sqrt(n2), 1e-8)

    s_blk = lax.dot_general(qn_ref[...], demb, (((1,), (1,)), ((), ())))

    bcols = lax.broadcasted_iota(jnp.int32, (_B, _NBLK), 1)
    bv, bi = _topk8(s_blk, bcols + i * _NBLK)
    for blk in range(_GRID):
        @pl.when(i == blk)
        def _(bv=bv, bi=bi, blk=blk):
            cvals_ref[:, pl.ds(blk * _K, _K)] = bv
            cidx_ref[:, pl.ds(blk * _K, _K)] = bi

    @pl.when(i == _GRID - 1)
    def _():
        ccols = lax.broadcasted_iota(jnp.int32, (_B, _NCAND), 1)
        work = cvals_ref[...]
        cid = cidx_ref[...]
        for k in range(_K):
            m = jnp.max(work, axis=1, keepdims=True)
            pos = jnp.min(jnp.where(work == m, ccols, _NCAND), axis=1,
                          keepdims=True)
            hit = ccols == pos
            vals_ref[:, pl.ds(k, 1)] = m
            idx_ref[:, pl.ds(k, 1)] = jnp.max(
                jnp.where(hit, cid, -1), axis=1, keepdims=True)
            work = jnp.where(hit, -jnp.inf, work)


def _retrieve_tc(da_query, da_documents, W_q, b_q, W_doc, b_doc):
    return pl.pallas_call(
        _tc_body,
        grid=(_GRID,),
        in_specs=[
            pl.BlockSpec((_NBLK, _S, _D), lambda i: (i, 0, 0)),
            pl.BlockSpec((_B, _D), lambda i: (0, 0)),
            pl.BlockSpec((_D, _D), lambda i: (0, 0)),
            pl.BlockSpec((1, _D), lambda i: (0, 0)),
            pl.BlockSpec((_D, _D), lambda i: (0, 0)),
            pl.BlockSpec((1, _D), lambda i: (0, 0)),
        ],
        out_specs=[
            pl.BlockSpec((_B, _K), lambda i: (0, 0)),
            pl.BlockSpec((_B, _K), lambda i: (0, 0)),
        ],
        out_shape=[
            jax.ShapeDtypeStruct((_B, _K), jnp.float32),
            jax.ShapeDtypeStruct((_B, _K), jnp.int32),
        ],
        scratch_shapes=[
            pltpu.VMEM((_B, _D), jnp.float32),
            pltpu.VMEM((_B, _NCAND), jnp.float32),
            pltpu.VMEM((_B, _NCAND), jnp.int32),
        ],
        compiler_params=pltpu.CompilerParams(
            dimension_semantics=("arbitrary",)),
    )(da_documents, da_query, W_q, b_q.reshape(1, _D), W_doc,
      b_doc.reshape(1, _D))


def _gather_docs_sc(docs, idx_flat):
    info = plsc.get_sparse_core_info()
    nc, ns = info.num_cores, info.num_subcores
    nw = nc * ns
    bpw = (_B * _K) // nw
    ch = 8
    mesh = plsc.VectorSubcoreMesh(core_axis_name="c", subcore_axis_name="s")

    @functools.partial(
        pl.kernel, mesh=mesh,
        out_type=jax.ShapeDtypeStruct((_B * _K, _S, _D), jnp.float32),
        scratch_types=[
            pltpu.VMEM((ch,), jnp.int32),
            pltpu.VMEM((ch, _S, _D), jnp.float32),
            pltpu.SemaphoreType.DMA,
        ],
    )
    def k(docs_hbm, idx_hbm, out_hbm, idx_v, rows_v, sem):
        wid = lax.axis_index("s") * nc + lax.axis_index("c")
        base = wid * bpw
        for c in range(bpw // ch):
            off = base + c * ch
            pltpu.sync_copy(idx_hbm.at[pl.ds(off, ch)], idx_v)
            pltpu.async_copy(docs_hbm.at[idx_v], rows_v, sem).wait()
            pltpu.sync_copy(rows_v, out_hbm.at[pl.ds(off, ch)])

    return k(docs, idx_flat)


def kernel(da_query, da_documents, W_q, b_q, W_doc, b_doc):
    top_vals, top_idx = _retrieve_tc(da_query, da_documents, W_q, b_q,
                                     W_doc, b_doc)
    rows = _gather_docs_sc(da_documents, top_idx.reshape(_B * _K))
    retrieved = rows.reshape(_B, _K, _S, _D)
    return retrieved, top_vals, top_idx


# speedup vs baseline: 1.2039x; 1.2039x over previous
import functools

import jax
import jax.numpy as jnp
from jax import lax
from jax.experimental import pallas as pl
from jax.experimental.pallas import tpu as pltpu
from jax.experimental.pallas import tpu_sc as plsc

_D = 1024
_N = 8192
_S = 8
_B = 64
_K = 8

_NBLK = 512
_GRID = _N // _NBLK
_NCAND = _GRID * _K


def _topk8(work, cols):
    vs, ids = [], []
    for k in range(_K):
        m = jnp.max(work, axis=1, keepdims=True)
        pos = jnp.min(jnp.where(work == m, cols, jnp.int32(2**30)),
                      axis=1, keepdims=True)
        vs.append(m)
        ids.append(pos)
        work = jnp.where(cols == pos, -jnp.inf, work)
    return jnp.concatenate(vs, axis=1), jnp.concatenate(ids, axis=1)


def _tc_body(docs_ref, q_ref, wq_ref, bq_ref, wdoc_ref, bdoc_ref,
             vals_ref, idx_ref, qn_ref, sims_ref):
    i = pl.program_id(0)

    @pl.when(i == 0)
    def _():
        q = lax.dot_general(q_ref[...], wq_ref[...],
                            (((1,), (1,)), ((), ()))) + bq_ref[...]
        n2 = jnp.sum(q * q, axis=1, keepdims=True)
        qn_ref[...] = q / jnp.maximum(jnp.sqrt(n2), 1e-8)

    avg = jnp.sum(docs_ref[...], axis=1) * (1.0 / _S)

    demb = lax.dot_general(avg, wdoc_ref[...],
                           (((1,), (1,)), ((), ()))) + bdoc_ref[...]
    n2 = jnp.sum(demb * demb, axis=1, keepdims=True)
    demb = demb / jnp.maximum(jnp.sqrt(n2), 1e-8)

    s_blk = lax.dot_general(qn_ref[...], demb, (((1,), (1,)), ((), ())))
    sims_ref[:, pl.ds(i * _NBLK, _NBLK)] = s_blk

    @pl.when(i == _GRID - 1)
    def _():
        work = sims_ref[...]
        cols = lax.broadcasted_iota(jnp.int32, (_B, _N), 1)
        for k in range(_K):
            m = jnp.max(work, axis=1, keepdims=True)
            idx = jnp.min(jnp.where(work == m, cols, _N), axis=1,
                          keepdims=True)
            vals_ref[:, pl.ds(k, 1)] = m
            idx_ref[:, pl.ds(k, 1)] = idx
            work = jnp.where(cols == idx, -jnp.inf, work)


def _retrieve_tc(da_query, da_documents, W_q, b_q, W_doc, b_doc):
    return pl.pallas_call(
        _tc_body,
        grid=(_GRID,),
        in_specs=[
            pl.BlockSpec((_NBLK, _S, _D), lambda i: (i, 0, 0)),
            pl.BlockSpec((_B, _D), lambda i: (0, 0)),
            pl.BlockSpec((_D, _D), lambda i: (0, 0)),
            pl.BlockSpec((1, _D), lambda i: (0, 0)),
            pl.BlockSpec((_D, _D), lambda i: (0, 0)),
            pl.BlockSpec((1, _D), lambda i: (0, 0)),
        ],
        out_specs=[
            pl.BlockSpec((_B, _K), lambda i: (0, 0)),
            pl.BlockSpec((_B, _K), lambda i: (0, 0)),
        ],
        out_shape=[
            jax.ShapeDtypeStruct((_B, _K), jnp.float32),
            jax.ShapeDtypeStruct((_B, _K), jnp.int32),
        ],
        scratch_shapes=[
            pltpu.VMEM((_B, _D), jnp.float32),
            pltpu.VMEM((_B, _N), jnp.float32),
        ],
        compiler_params=pltpu.CompilerParams(
            dimension_semantics=("arbitrary",)),
    )(da_documents, da_query, W_q, b_q.reshape(1, _D), W_doc,
      b_doc.reshape(1, _D))


def _gather_docs_sc(docs, idx_flat):
    info = plsc.get_sparse_core_info()
    nc, ns = info.num_cores, info.num_subcores
    nw = nc * ns
    bpw = (_B * _K) // nw
    ch = 8
    mesh = plsc.VectorSubcoreMesh(core_axis_name="c", subcore_axis_name="s")

    @functools.partial(
        pl.kernel, mesh=mesh,
        out_type=jax.ShapeDtypeStruct((_B * _K, _S, _D), jnp.float32),
        scratch_types=[
            pltpu.VMEM((ch,), jnp.int32),
            pltpu.VMEM((ch, _S, _D), jnp.float32),
            pltpu.SemaphoreType.DMA,
        ],
    )
    def k(docs_hbm, idx_hbm, out_hbm, idx_v, rows_v, sem):
        wid = lax.axis_index("s") * nc + lax.axis_index("c")
        base = wid * bpw
        for c in range(bpw // ch):
            off = base + c * ch
            pltpu.sync_copy(idx_hbm.at[pl.ds(off, ch)], idx_v)
            pltpu.async_copy(docs_hbm.at[idx_v], rows_v, sem).wait()
            pltpu.sync_copy(rows_v, out_hbm.at[pl.ds(off, ch)])

    return k(docs, idx_flat)


def kernel(da_query, da_documents, W_q, b_q, W_doc, b_doc):
    top_vals, top_idx = _retrieve_tc(da_query, da_documents, W_q, b_q,
                                     W_doc, b_doc)
    rows = _gather_docs_sc(da_documents, top_idx.reshape(_B * _K))
    retrieved = rows.reshape(_B, _K, _S, _D)
    return retrieved, top_vals, top_idx


# speedup vs baseline: 1.3175x; 1.0943x over previous
import functools

import jax
import jax.numpy as jnp
from jax import lax
from jax.experimental import pallas as pl
from jax.experimental.pallas import tpu as pltpu
from jax.experimental.pallas import tpu_sc as plsc

_D = 1024
_N = 8192
_S = 8
_B = 64
_K = 8

_NBLK = 512
_GRID = _N // _NBLK


def _topk8(work, cols):
    vs, ids = [], []
    for k in range(_K):
        m = jnp.max(work, axis=1, keepdims=True)
        pos = jnp.min(jnp.where(work == m, cols, jnp.int32(2**30)),
                      axis=1, keepdims=True)
        vs.append(m)
        ids.append(pos)
        work = jnp.where(cols == pos, -jnp.inf, work)
    return jnp.concatenate(vs, axis=1), jnp.concatenate(ids, axis=1)


def _tc_body(docs_ref, q_ref, wq_ref, bq_ref, wdoc_ref, bdoc_ref,
             vals_ref, idx_ref, qn_ref, sims_ref):
    i = pl.program_id(0)

    @pl.when(i == 0)
    def _():
        q = lax.dot_general(q_ref[...], wq_ref[...],
                            (((1,), (1,)), ((), ()))) + bq_ref[...]
        n2 = jnp.sum(q * q, axis=1, keepdims=True)
        qn_ref[...] = q / jnp.maximum(jnp.sqrt(n2), 1e-8)

    parts = [docs_ref[:, s, :] for s in range(_S)]
    while len(parts) > 1:
        parts = [parts[j] + parts[j + 1] for j in range(0, len(parts), 2)]
    avg = parts[0] * (1.0 / _S)

    demb = lax.dot_general(avg, wdoc_ref[...],
                           (((1,), (1,)), ((), ()))) + bdoc_ref[...]
    n2 = jnp.sum(demb * demb, axis=1, keepdims=True)
    inv = jax.lax.rsqrt(jnp.maximum(n2, 1e-16))

    s_blk = lax.dot_general(qn_ref[...], demb, (((1,), (1,)), ((), ())))
    s_blk = s_blk * jnp.reshape(inv, (1, _NBLK))
    sims_ref[:, pl.ds(i * _NBLK, _NBLK)] = s_blk

    @pl.when(i == _GRID - 1)
    def _():
        work = sims_ref[...]
        cols = lax.broadcasted_iota(jnp.int32, (_B, _N), 1)
        for k in range(_K):
            m = jnp.max(work, axis=1, keepdims=True)
            idx = jnp.min(jnp.where(work == m, cols, _N), axis=1,
                          keepdims=True)
            vals_ref[:, pl.ds(k, 1)] = m
            idx_ref[:, pl.ds(k, 1)] = idx
            work = jnp.where(cols == idx, -jnp.inf, work)


def _retrieve_tc(da_query, da_documents, W_q, b_q, W_doc, b_doc):
    return pl.pallas_call(
        _tc_body,
        grid=(_GRID,),
        in_specs=[
            pl.BlockSpec((_NBLK, _S, _D), lambda i: (i, 0, 0)),
            pl.BlockSpec((_B, _D), lambda i: (0, 0)),
            pl.BlockSpec((_D, _D), lambda i: (0, 0)),
            pl.BlockSpec((1, _D), lambda i: (0, 0)),
            pl.BlockSpec((_D, _D), lambda i: (0, 0)),
            pl.BlockSpec((1, _D), lambda i: (0, 0)),
        ],
        out_specs=[
            pl.BlockSpec((_B, _K), lambda i: (0, 0)),
            pl.BlockSpec((_B, _K), lambda i: (0, 0)),
        ],
        out_shape=[
            jax.ShapeDtypeStruct((_B, _K), jnp.float32),
            jax.ShapeDtypeStruct((_B, _K), jnp.int32),
        ],
        scratch_shapes=[
            pltpu.VMEM((_B, _D), jnp.float32),
            pltpu.VMEM((_B, _N), jnp.float32),
        ],
        compiler_params=pltpu.CompilerParams(
            dimension_semantics=("arbitrary",),
            vmem_limit_bytes=100 * 1024 * 1024),
    )(da_documents, da_query, W_q, b_q.reshape(1, _D), W_doc,
      b_doc.reshape(1, _D))


def _gather_docs_sc(docs, idx_flat):
    info = plsc.get_sparse_core_info()
    nc, ns = info.num_cores, info.num_subcores
    nw = nc * ns
    bpw = (_B * _K) // nw
    ch = 8
    mesh = plsc.VectorSubcoreMesh(core_axis_name="c", subcore_axis_name="s")

    @functools.partial(
        pl.kernel, mesh=mesh,
        out_type=jax.ShapeDtypeStruct((_B * _K, _S, _D), jnp.float32),
        scratch_types=[
            pltpu.VMEM((ch,), jnp.int32),
            pltpu.VMEM((ch, _S, _D), jnp.float32),
            pltpu.SemaphoreType.DMA,
        ],
    )
    def k(docs_hbm, idx_hbm, out_hbm, idx_v, rows_v, sem):
        wid = lax.axis_index("s") * nc + lax.axis_index("c")
        base = wid * bpw
        for c in range(bpw // ch):
            off = base + c * ch
            pltpu.sync_copy(idx_hbm.at[pl.ds(off, ch)], idx_v)
            pltpu.async_copy(docs_hbm.at[idx_v], rows_v, sem).wait()
            pltpu.sync_copy(rows_v, out_hbm.at[pl.ds(off, ch)])

    return k(docs, idx_flat)


def kernel(da_query, da_documents, W_q, b_q, W_doc, b_doc):
    top_vals, top_idx = _retrieve_tc(da_query, da_documents, W_q, b_q,
                                     W_doc, b_doc)
    rows = _gather_docs_sc(da_documents, top_idx.reshape(_B * _K))
    retrieved = rows.reshape(_B, _K, _S, _D)
    return retrieved, top_vals, top_idx
